# Initial kernel scaffold; baseline (speedup 1.0000x reference)
import jax
import jax.numpy as jnp
from jax.experimental import pallas as pl


def kernel(F, patient_edge_index, feat_edge_index, W_fg, as_fg, ad_fg, b_fg, W_pg, as_pg, ad_pg, b_pg, W1, as1, ad1, b1, Wc, bc):
    raise NotImplementedError("write your pallas kernel here")


# speedup vs baseline: 2.0216x; 2.0216x over previous
import functools

import jax
import jax.numpy as jnp
from jax import lax
from jax.experimental import pallas as pl
from jax.experimental.pallas import tpu as pltpu

N = 4096
D = 4096
H = 512
C = 32

_MBLK = 512


def _matvecs_body(f_ref, wfg_ref, wpg_ref, hf_ref, hp_ref):
    i = pl.program_id(0)
    fb = f_ref[...]
    hp_ref[...] = jnp.dot(fb, wpg_ref[...], preferred_element_type=jnp.float32)
    part = jnp.dot(wfg_ref[...], fb, preferred_element_type=jnp.float32)

    @pl.when(i == 0)
    def _():
        hf_ref[...] = part

    @pl.when(i > 0)
    def _():
        hf_ref[...] += part


def _tc_matvecs(F, W_fg_row, W_pg_col):
    nblk = N // _MBLK
    return pl.pallas_call(
        _matvecs_body,
        grid=(nblk,),
        in_specs=[
            pl.BlockSpec((_MBLK, D), lambda i: (i, 0)),
            pl.BlockSpec((1, _MBLK), lambda i: (0, i)),
            pl.BlockSpec((D, 1), lambda i: (0, 0)),
        ],
        out_specs=[
            pl.BlockSpec((1, D), lambda i: (0, 0)),
            pl.BlockSpec((_MBLK, 1), lambda i: (i, 0)),
        ],
        out_shape=[
            jax.ShapeDtypeStruct((1, D), jnp.float32),
            jax.ShapeDtypeStruct((N, 1), jnp.float32),
        ],
    )(F, W_fg_row, W_pg_col)


def _main_mm_body(f_ref, wf_ref, wp_ref, w1_ref, as_ref, ad_ref,
                  h1_ref, as_out_ref, ad_out_ref):
    x = f_ref[...] * wf_ref[...] * wp_ref[...]
    h1 = jnp.dot(x, w1_ref[...], preferred_element_type=jnp.float32)
    h1_ref[...] = h1
    as_out_ref[...] = jnp.dot(h1, as_ref[...], preferred_element_type=jnp.float32)
    ad_out_ref[...] = jnp.dot(h1, ad_ref[...], preferred_element_type=jnp.float32)


def _tc_main_matmul(F, W_f_row, W_p_col, W1, as1_col, ad1_col):
    nblk = N // _MBLK
    return pl.pallas_call(
        _main_mm_body,
        grid=(nblk,),
        in_specs=[
            pl.BlockSpec((_MBLK, D), lambda i: (i, 0)),
            pl.BlockSpec((1, D), lambda i: (0, 0)),
            pl.BlockSpec((_MBLK, 1), lambda i: (i, 0)),
            pl.BlockSpec((D, H), lambda i: (0, 0)),
            pl.BlockSpec((H, 1), lambda i: (0, 0)),
            pl.BlockSpec((H, 1), lambda i: (0, 0)),
        ],
        out_specs=[
            pl.BlockSpec((_MBLK, H), lambda i: (i, 0)),
            pl.BlockSpec((_MBLK, 1), lambda i: (i, 0)),
            pl.BlockSpec((_MBLK, 1), lambda i: (i, 0)),
        ],
        out_shape=[
            jax.ShapeDtypeStruct((N, H), jnp.float32),
            jax.ShapeDtypeStruct((N, 1), jnp.float32),
            jax.ShapeDtypeStruct((N, 1), jnp.float32),
        ],
    )(F, W_f_row, W_p_col, W1, as1_col, ad1_col)


def _final_body(num_ref, den_ref, b1_ref, wc_ref, bc_ref, out_ref):
    x = num_ref[...] / (den_ref[...] + 1e-16) + b1_ref[...]
    x = jnp.maximum(x, 0.0)
    out_ref[...] = (
        jnp.dot(x, wc_ref[...], preferred_element_type=jnp.float32)
        + bc_ref[...]
    )


def _tc_final(num, den_col, b1_row, Wc, bc_row):
    return pl.pallas_call(
        _final_body,
        in_specs=[
            pl.BlockSpec((N, H), lambda: (0, 0)),
            pl.BlockSpec((N, 1), lambda: (0, 0)),
            pl.BlockSpec((1, H), lambda: (0, 0)),
            pl.BlockSpec((H, C), lambda: (0, 0)),
            pl.BlockSpec((1, C), lambda: (0, 0)),
        ],
        out_specs=pl.BlockSpec((N, C), lambda: (0, 0)),
        out_shape=jax.ShapeDtypeStruct((N, C), jnp.float32),
    )(num, den_col, b1_row, Wc, bc_row)


def _edge_softmax_scalar(h, src, dst, a_s, a_d, num_nodes):
    alpha = a_s * h[src] + a_d * h[dst]
    alpha = jnp.where(alpha > 0, alpha, 0.2 * alpha)
    e = jnp.exp(alpha)
    den = jax.ops.segment_sum(e, dst, num_segments=num_nodes)
    num = jax.ops.segment_sum(e * h[src], dst, num_segments=num_nodes)
    return num, den


def _edge_softmax_vector(h1, a_s, a_d, src, dst, num_nodes):
    alpha = a_s[src] + a_d[dst]
    alpha = jnp.where(alpha > 0, alpha, 0.2 * alpha)
    e = jnp.exp(alpha)
    den = jax.ops.segment_sum(e, dst, num_segments=num_nodes)
    num = jax.ops.segment_sum(e[:, None] * h1[src], dst, num_segments=num_nodes)
    return num, den


def kernel(F, patient_edge_index, feat_edge_index,
           W_fg, as_fg, ad_fg, b_fg,
           W_pg, as_pg, ad_pg, b_pg,
           W1, as1, ad1, b1,
           Wc, bc):
    loop_n = jnp.arange(N, dtype=jnp.int32)
    loop_d = jnp.arange(D, dtype=jnp.int32)
    p_src = jnp.concatenate([patient_edge_index[0].astype(jnp.int32), loop_n])
    p_dst = jnp.concatenate([patient_edge_index[1].astype(jnp.int32), loop_n])
    f_src = jnp.concatenate([feat_edge_index[0].astype(jnp.int32), loop_d])
    f_dst = jnp.concatenate([feat_edge_index[1].astype(jnp.int32), loop_d])

    hf_row, hp_col = _tc_matvecs(F, W_fg.reshape(1, N), W_pg.reshape(D, 1))
    hf = hf_row.reshape(D)
    hp = hp_col.reshape(N)

    f_num, f_den = _edge_softmax_scalar(hf, f_src, f_dst,
                                        as_fg[0], ad_fg[0], D)
    W_f = f_num / (f_den + 1e-16) + b_fg[0]
    p_num, p_den = _edge_softmax_scalar(hp, p_src, p_dst,
                                        as_pg[0], ad_pg[0], N)
    W_p = p_num / (p_den + 1e-16) + b_pg[0]

    h1, as1_col, ad1_col = _tc_main_matmul(
        F, W_f.reshape(1, D), W_p.reshape(N, 1),
        W1, as1.reshape(H, 1), ad1.reshape(H, 1))

    num, den = _edge_softmax_vector(h1, as1_col.reshape(N), ad1_col.reshape(N),
                                    p_src, p_dst, N)

    return _tc_final(num, den.reshape(N, 1), b1.reshape(1, H),
                     Wc, bc.reshape(1, C))


# speedup vs baseline: 4.9057x; 2.4267x over previous
import functools

import jax
import jax.numpy as jnp
from jax import lax
from jax.experimental import pallas as pl
from jax.experimental.pallas import tpu as pltpu
from jax.experimental.pallas import tpu_sc as plsc

N = 4096
D = 4096
H = 512
C = 32

_MBLK = 512

_NS = 16
_ET = 65536 + 4096
_EPT = _ET // _NS
_ECH = 128
_NCH = _EPT // _ECH


def _matvecs_body(f_ref, wfg_ref, wpg_ref, hf_ref, hp_ref):
    i = pl.program_id(0)
    fb = f_ref[...]
    hp_ref[...] = jnp.dot(fb, wpg_ref[...], preferred_element_type=jnp.float32)
    part = jnp.dot(wfg_ref[...], fb, preferred_element_type=jnp.float32)

    @pl.when(i == 0)
    def _():
        hf_ref[...] = part

    @pl.when(i > 0)
    def _():
        hf_ref[...] += part


def _tc_matvecs(F, W_fg_row, W_pg_col):
    nblk = N // _MBLK
    return pl.pallas_call(
        _matvecs_body,
        grid=(nblk,),
        in_specs=[
            pl.BlockSpec((_MBLK, D), lambda i: (i, 0)),
            pl.BlockSpec((1, _MBLK), lambda i: (0, i)),
            pl.BlockSpec((D, 1), lambda i: (0, 0)),
        ],
        out_specs=[
            pl.BlockSpec((1, D), lambda i: (0, 0)),
            pl.BlockSpec((_MBLK, 1), lambda i: (i, 0)),
        ],
        out_shape=[
            jax.ShapeDtypeStruct((1, D), jnp.float32),
            jax.ShapeDtypeStruct((N, 1), jnp.float32),
        ],
    )(F, W_fg_row, W_pg_col)


def _main_mm_body(f_ref, wf_ref, wp_ref, w1_ref, as_ref, ad_ref,
                  h1_ref, as_out_ref, ad_out_ref):
    x = f_ref[...] * wf_ref[...] * wp_ref[...]
    h1 = jnp.dot(x, w1_ref[...], preferred_element_type=jnp.float32)
    h1_ref[...] = h1
    as_out_ref[...] = jnp.dot(h1, as_ref[...], preferred_element_type=jnp.float32)
    ad_out_ref[...] = jnp.dot(h1, ad_ref[...], preferred_element_type=jnp.float32)


def _tc_main_matmul(F, W_f_row, W_p_col, W1, as1_col, ad1_col):
    nblk = N // _MBLK
    return pl.pallas_call(
        _main_mm_body,
        grid=(nblk,),
        in_specs=[
            pl.BlockSpec((_MBLK, D), lambda i: (i, 0)),
            pl.BlockSpec((1, D), lambda i: (0, 0)),
            pl.BlockSpec((_MBLK, 1), lambda i: (i, 0)),
            pl.BlockSpec((D, H), lambda i: (0, 0)),
            pl.BlockSpec((H, 1), lambda i: (0, 0)),
            pl.BlockSpec((H, 1), lambda i: (0, 0)),
        ],
        out_specs=[
            pl.BlockSpec((_MBLK, H), lambda i: (i, 0)),
            pl.BlockSpec((_MBLK, 1), lambda i: (i, 0)),
            pl.BlockSpec((_MBLK, 1), lambda i: (i, 0)),
        ],
        out_shape=[
            jax.ShapeDtypeStruct((N, H), jnp.float32),
            jax.ShapeDtypeStruct((N, 1), jnp.float32),
            jax.ShapeDtypeStruct((N, 1), jnp.float32),
        ],
    )(F, W_f_row, W_p_col, W1, as1_col, ad1_col)


def _final_body(num_ref, den_ref, b1_ref, wc_ref, bc_ref, out_ref):
    x = num_ref[...] / (den_ref[...] + 1e-16) + b1_ref[...]
    x = jnp.maximum(x, 0.0)
    out_ref[...] = (
        jnp.dot(x, wc_ref[...], preferred_element_type=jnp.float32)
        + bc_ref[...]
    )


def _tc_final(num, den_col, b1_row, Wc, bc_row):
    return pl.pallas_call(
        _final_body,
        in_specs=[
            pl.BlockSpec((N, H), lambda: (0, 0)),
            pl.BlockSpec((N, 1), lambda: (0, 0)),
            pl.BlockSpec((1, H), lambda: (0, 0)),
            pl.BlockSpec((H, C), lambda: (0, 0)),
            pl.BlockSpec((1, C), lambda: (0, 0)),
        ],
        out_specs=pl.BlockSpec((N, C), lambda: (0, 0)),
        out_shape=jax.ShapeDtypeStruct((N, C), jnp.float32),
    )(num, den_col, b1_row, Wc, bc_row)


def _sc_small_body(h2, asad, srcg, dstg, zden,
                   num_out, den_out,
                   h_v, as_v, ad_v, src_v, dst_v, x_v, xh_v,
                   num_sh, den_sh):
    c = lax.axis_index("c")
    s = lax.axis_index("s")
    pltpu.sync_copy(h2.at[c], h_v)
    pltpu.sync_copy(asad.at[c, 0], as_v)
    pltpu.sync_copy(asad.at[c, 1], ad_v)
    pltpu.sync_copy(srcg.at[c, s], src_v)
    pltpu.sync_copy(dstg.at[c, s], dst_v)

    @pl.when(s == 0)
    def _():
        pltpu.sync_copy(zden, num_sh)
        pltpu.sync_copy(zden, den_sh)

    plsc.subcore_barrier()

    asv = as_v[...]
    adv = ad_v[...]

    def chunk(ci, carry):
        def lane(j, carry2):
            s16 = src_v[ci, pl.ds(j * 16, 16)]
            d16 = dst_v[ci, pl.ds(j * 16, 16)]
            hs = plsc.load_gather(h_v, [s16])
            hd = plsc.load_gather(h_v, [d16])
            al = asv * hs + adv * hd
            al = jnp.where(al > 0, al, 0.2 * al)
            x = jnp.exp(al)
            x_v[ci, pl.ds(j * 16, 16)] = x
            xh_v[ci, pl.ds(j * 16, 16)] = x * hs
            return carry2
        return lax.fori_loop(0, _ECH // 16, lane, carry)

    lax.fori_loop(0, _NCH, chunk, 0)

    def scat(ci, carry):
        pltpu.sync_copy(x_v.at[ci], den_sh.at[dst_v.at[ci]], add=True)
        pltpu.sync_copy(xh_v.at[ci], num_sh.at[dst_v.at[ci]], add=True)
        return carry

    lax.fori_loop(0, _NCH, scat, 0)
    plsc.subcore_barrier()

    off = s * (N // _NS)
    pltpu.sync_copy(num_sh.at[pl.ds(off, N // _NS)],
                    num_out.at[c, pl.ds(off, N // _NS)])
    pltpu.sync_copy(den_sh.at[pl.ds(off, N // _NS)],
                    den_out.at[c, pl.ds(off, N // _NS)])


@functools.cache
def _sc_small():
    return pl.kernel(
        _sc_small_body,
        mesh=plsc.VectorSubcoreMesh(core_axis_name="c", subcore_axis_name="s"),
        compiler_params=pltpu.CompilerParams(needs_layout_passes=False),
        out_type=[
            jax.ShapeDtypeStruct((2, N), jnp.float32),
            jax.ShapeDtypeStruct((2, N), jnp.float32),
        ],
        scratch_types=[
            pltpu.VMEM((N,), jnp.float32),
            pltpu.VMEM((16,), jnp.float32),
            pltpu.VMEM((16,), jnp.float32),
            pltpu.VMEM((_NCH, _ECH), jnp.int32),
            pltpu.VMEM((_NCH, _ECH), jnp.int32),
            pltpu.VMEM((_NCH, _ECH), jnp.float32),
            pltpu.VMEM((_NCH, _ECH), jnp.float32),
            pltpu.VMEM_SHARED((N,), jnp.float32),
            pltpu.VMEM_SHARED((N,), jnp.float32),
        ],
    )


def _edge_softmax_vector(h1, a_s, a_d, src, dst, num_nodes):
    alpha = a_s[src] + a_d[dst]
    alpha = jnp.where(alpha > 0, alpha, 0.2 * alpha)
    e = jnp.exp(alpha)
    den = jax.ops.segment_sum(e, dst, num_segments=num_nodes)
    num = jax.ops.segment_sum(e[:, None] * h1[src], dst, num_segments=num_nodes)
    return num, den


def kernel(F, patient_edge_index, feat_edge_index,
           W_fg, as_fg, ad_fg, b_fg,
           W_pg, as_pg, ad_pg, b_pg,
           W1, as1, ad1, b1,
           Wc, bc):
    loop_n = jnp.arange(N, dtype=jnp.int32)
    loop_d = jnp.arange(D, dtype=jnp.int32)
    p_src = jnp.concatenate([patient_edge_index[0].astype(jnp.int32), loop_n])
    p_dst = jnp.concatenate([patient_edge_index[1].astype(jnp.int32), loop_n])
    f_src = jnp.concatenate([feat_edge_index[0].astype(jnp.int32), loop_d])
    f_dst = jnp.concatenate([feat_edge_index[1].astype(jnp.int32), loop_d])

    hf_row, hp_col = _tc_matvecs(F, W_fg.reshape(1, N), W_pg.reshape(D, 1))
    hf = hf_row.reshape(D)
    hp = hp_col.reshape(N)

    h2 = jnp.stack([hf, hp])
    ones16 = jnp.ones((16,), jnp.float32)
    asad = jnp.stack([
        jnp.stack([as_fg[0] * ones16, ad_fg[0] * ones16]),
        jnp.stack([as_pg[0] * ones16, ad_pg[0] * ones16]),
    ])
    srcg = jnp.stack([f_src, p_src]).reshape(2, _NS, _NCH, _ECH)
    dstg = jnp.stack([f_dst, p_dst]).reshape(2, _NS, _NCH, _ECH)
    zden = jnp.zeros((N,), jnp.float32)
    num2, den2 = _sc_small()(h2, asad, srcg, dstg, zden)
    W_f = num2[0] / (den2[0] + 1e-16) + b_fg[0]
    W_p = num2[1] / (den2[1] + 1e-16) + b_pg[0]

    h1, as1_col, ad1_col = _tc_main_matmul(
        F, W_f.reshape(1, D), W_p.reshape(N, 1),
        W1, as1.reshape(H, 1), ad1.reshape(H, 1))

    num, den = _edge_softmax_vector(h1, as1_col.reshape(N), ad1_col.reshape(N),
                                    p_src, p_dst, N)

    return _tc_final(num, den.reshape(N, 1), b1.reshape(1, H),
                     Wc, bc.reshape(1, C))
